# baseline (speedup 1.0000x reference)
import functools

import jax
import jax.numpy as jnp
from jax import lax
from jax.experimental import pallas as pl
from jax.experimental.pallas import tpu as pltpu
from jax.experimental.pallas import tpu_sc as plsc

N_V = 10000
N_E = 10000
NNZ = 320000
D = 128

_NC = 2
_NS = 16
_NW = _NC * _NS
_K = 40
_PPW = NNZ // _NW
_PPC = NNZ // _NS
_CH = NNZ // _NW // _K
_CCH = NNZ // _NS // _K
_NP = 10240
_RPT = _NP // _NS

_mesh = plsc.VectorSubcoreMesh(core_axis_name="c", subcore_axis_name="s")


_NBUF = 5
assert _CH % _NBUF == 0


def _sc_body_gs(table, src1d, dst1d, zeros, out, sidx, didx,
                rows0, rows1, rows2, rows3, rows4,
                acc, gs0, gs1, gs2, gs3, gs4, ss0, ss1, ss2, ss3, ss4):
    rows = (rows0, rows1, rows2, rows3, rows4)
    gsems = (gs0, gs1, gs2, gs3, gs4)
    ssems = (ss0, ss1, ss2, ss3, ss4)
    c = lax.axis_index("c")
    s = lax.axis_index("s")
    wid = s * _NC + c
    pltpu.sync_copy(src1d.at[pl.ds(wid * _PPW, _PPW)], sidx)
    pltpu.sync_copy(dst1d.at[pl.ds(wid * _PPW, _PPW)], didx)
    pltpu.sync_copy(zeros.at[pl.ds(s * _RPT, _RPT)], acc.at[pl.ds(s * _RPT, _RPT)])
    plsc.subcore_barrier()

    def fire_g(j, b):
        pltpu.async_copy(table.at[sidx.at[pl.ds(j * _K, _K)]], rows[b], gsems[b])

    def wait_g(j, b):
        pltpu.make_async_copy(table.at[sidx.at[pl.ds(j * _K, _K)]],
                              rows[b], gsems[b]).wait()

    def fire_s(j, b):
        pltpu.async_copy(rows[b], acc.at[didx.at[pl.ds(j * _K, _K)]],
                         ssems[b], add=True)

    def wait_s(b):
        pltpu.make_async_copy(rows[b], acc.at[didx.at[pl.ds(0, _K)]],
                              ssems[b]).wait()

    def slot(t, b, fire_ahead=True, wait_behind=True):
        wait_g(t, b)
        fire_s(t, b)
        if wait_behind:
            wait_s((b + 3) % _NBUF)
        if fire_ahead:
            fire_g(t + 2, (b + 2) % _NBUF)

    fire_g(0, 0)
    fire_g(1, 1)
    slot(0, 0, wait_behind=False)
    slot(1, 1, wait_behind=False)
    slot(2, 2)
    slot(3, 3)
    slot(4, 4)

    @pl.loop(_NBUF, _CH - _NBUF, step=_NBUF)
    def _(j):
        for dt in range(_NBUF):
            slot(j + dt, dt)

    for dt in range(3):
        slot(_CH - _NBUF + dt, dt)
    for dt in range(3, _NBUF):
        slot(_CH - _NBUF + dt, dt, fire_ahead=False)
    wait_s(3)
    wait_s(4)

    plsc.subcore_barrier()
    pltpu.sync_copy(acc.at[pl.ds(s * _RPT, _RPT)],
                    out.at[pl.ds(c * _NP + s * _RPT, _RPT)])


_sc_gather_scatter = functools.partial(
    pl.kernel,
    out_type=jax.ShapeDtypeStruct((2 * _NP, D), jnp.float32),
    mesh=_mesh,
    scratch_types=(
        [pltpu.VMEM((_PPW,), jnp.int32),
         pltpu.VMEM((_PPW,), jnp.int32)]
        + [pltpu.VMEM((_K, D), jnp.float32) for _ in range(_NBUF)]
        + [pltpu.VMEM_SHARED((_NP, D), jnp.float32)]
        + [pltpu.SemaphoreType.DMA for _ in range(2 * _NBUF)]
    ),
)(_sc_body_gs)


_KC = 200
_CCC = _PPC // _KC
_CW = 4


def _sc_body_cnt(idxcat, ones, zeros, out, cidx, ones_v, acc, csem):
    c = lax.axis_index("c")
    s = lax.axis_index("s")
    pltpu.sync_copy(idxcat.at[pl.ds(c * NNZ + s * _PPC, _PPC)], cidx)
    pltpu.sync_copy(ones, ones_v)
    pltpu.sync_copy(zeros.at[pl.ds(s * _RPT, _RPT)], acc.at[pl.ds(s * _RPT, _RPT)])
    plsc.subcore_barrier()

    def fire(j):
        pltpu.async_copy(ones_v, acc.at[cidx.at[pl.ds(j * _KC, _KC)]],
                         csem, add=True)

    def drain_one():
        pltpu.make_async_copy(ones_v, acc.at[cidx.at[pl.ds(0, _KC)]],
                              csem).wait()

    for w in range(_CW):
        fire(w)

    @pl.loop(_CW, _CCC)
    def _(j):
        drain_one()
        fire(j)

    for _w in range(_CW):
        drain_one()

    plsc.subcore_barrier()
    pltpu.sync_copy(acc.at[pl.ds(s * _RPT, _RPT)],
                    out.at[pl.ds(c * _NP + s * _RPT, _RPT)])


_sc_counts = functools.partial(
    pl.kernel,
    out_type=jax.ShapeDtypeStruct((2 * _NP, D), jnp.float32),
    mesh=_mesh,
    scratch_types=[
        pltpu.VMEM((_PPC,), jnp.int32),
        pltpu.VMEM((_KC, D), jnp.float32),
        pltpu.VMEM_SHARED((_NP, D), jnp.float32),
        pltpu.SemaphoreType.DMA,
    ],
)(_sc_body_cnt)


_R = 1000
_RP = 1024


def _tc_mm(x, w, b2d):
    def body(x_ref, w_ref, b_ref, o_ref):
        o_ref[...] = (jnp.dot(x_ref[...], w_ref[...],
                              preferred_element_type=jnp.float32) + b_ref[...])

    return pl.pallas_call(
        body,
        grid=(N_V // _R,),
        in_specs=[
            pl.BlockSpec((_R, D), lambda i: (i, 0)),
            pl.BlockSpec((D, D), lambda i: (0, 0)),
            pl.BlockSpec((1, D), lambda i: (0, 0)),
        ],
        out_specs=pl.BlockSpec((_R, D), lambda i: (i, 0)),
        out_shape=jax.ShapeDtypeStruct((N_V, D), jnp.float32),
    )(x, w, b2d)


def _tc_combine(partials, cnts, off):
    def body(p0_ref, p1_ref, c_ref, o_ref):
        cnt = jnp.maximum(c_ref[...][:, :1], 1.0)
        o_ref[...] = (p0_ref[...] + p1_ref[...]) / cnt

    nb = _NP // _RP
    return pl.pallas_call(
        body,
        grid=(nb,),
        in_specs=[
            pl.BlockSpec((_RP, D), lambda i: (i, 0)),
            pl.BlockSpec((_RP, D), lambda i: (i + nb, 0)),
            pl.BlockSpec((_RP, D), lambda i: (i + off, 0)),
        ],
        out_specs=pl.BlockSpec((_RP, D), lambda i: (i, 0)),
        out_shape=jax.ShapeDtypeStruct((_NP, D), jnp.float32),
    )(partials, partials, cnts)


def _tc_combine_relu_mm(partials, cnts, off, w, b2d):
    def body(p0_ref, p1_ref, c_ref, w_ref, b_ref, o_ref):
        cnt = jnp.maximum(c_ref[...][:, :1], 1.0)
        v = (p0_ref[...] + p1_ref[...]) / cnt
        v = jnp.maximum(v, 0.0)
        o_ref[...] = (jnp.dot(v, w_ref[...],
                              preferred_element_type=jnp.float32) + b_ref[...])

    nb = _NP // _RP
    return pl.pallas_call(
        body,
        grid=(nb,),
        in_specs=[
            pl.BlockSpec((_RP, D), lambda i: (i, 0)),
            pl.BlockSpec((_RP, D), lambda i: (i + nb, 0)),
            pl.BlockSpec((_RP, D), lambda i: (i + off, 0)),
            pl.BlockSpec((D, D), lambda i: (0, 0)),
            pl.BlockSpec((1, D), lambda i: (0, 0)),
        ],
        out_specs=pl.BlockSpec((_RP, D), lambda i: (i, 0)),
        out_shape=jax.ShapeDtypeStruct((_NP, D), jnp.float32),
    )(partials, partials, cnts, w, b2d)


def kernel(X, incidence, W1, b1, W2, b2):
    v_idx = incidence[0].astype(jnp.int32)
    e_idx = incidence[1].astype(jnp.int32)
    idxcat = jnp.concatenate([e_idx, v_idx])
    zeros = jnp.zeros((_NP, D), jnp.float32)
    ones = jnp.ones((_KC, D), jnp.float32)
    b1r = b1.reshape(1, D)
    b2r = b2.reshape(1, D)

    nb = _NP // _RP
    cnt = _sc_counts(idxcat, ones, zeros)
    h1 = _tc_mm(X, W1, b1r)
    p = _sc_gather_scatter(h1, v_idx, e_idx, zeros)
    he1 = _tc_combine(p, cnt, 0)
    q = _sc_gather_scatter(he1, e_idx, v_idx, zeros)
    h2 = _tc_combine_relu_mm(q, cnt, nb, W2, b2r)
    p2 = _sc_gather_scatter(h2, v_idx, e_idx, zeros)
    he2 = _tc_combine(p2, cnt, 0)
    q2 = _sc_gather_scatter(he2, e_idx, v_idx, zeros)
    return _tc_combine(q2, cnt, nb)[:N_V]


# speedup vs baseline: 1.2559x; 1.2559x over previous
import functools

import jax
import jax.numpy as jnp
from jax import lax
from jax.experimental import pallas as pl
from jax.experimental.pallas import tpu as pltpu
from jax.experimental.pallas import tpu_sc as plsc

N_V = 10000
N_E = 10000
NNZ = 320000
D = 128

_NC = 2
_NS = 16
_NW = _NC * _NS
_K = 40
_PPW = NNZ // _NW
_PPC = NNZ // _NS
_CH = NNZ // _NW // _K
_CCH = NNZ // _NS // _K
_NP = 10240
_RPT = _NP // _NS

_mesh = plsc.VectorSubcoreMesh(core_axis_name="c", subcore_axis_name="s")


_NBUF = 4
_TAIL = _CH % _NBUF
assert 1 <= _TAIL <= 2


def _sc_body_gs(table, src1d, dst1d, zeros, out, sidx, didx,
                rows0, rows1, rows2, rows3, acc, sem0, sem1, sem2, sem3):
    rows = (rows0, rows1, rows2, rows3)
    sems = (sem0, sem1, sem2, sem3)
    c = lax.axis_index("c")
    s = lax.axis_index("s")
    wid = s * _NC + c
    pltpu.sync_copy(src1d.at[pl.ds(wid * _PPW, _PPW)], sidx)
    pltpu.sync_copy(dst1d.at[pl.ds(wid * _PPW, _PPW)], didx)
    pltpu.sync_copy(zeros.at[pl.ds(s * _RPT, _RPT)], acc.at[pl.ds(s * _RPT, _RPT)])
    plsc.subcore_barrier()

    def gather(j, b):
        pltpu.async_copy(table.at[sidx.at[pl.ds(j * _K, _K)]], rows[b], sems[b])

    def wait_scatter(j, b):
        pltpu.make_async_copy(table.at[sidx.at[pl.ds(j * _K, _K)]],
                              rows[b], sems[b]).wait()
        pltpu.sync_copy(rows[b], acc.at[didx.at[pl.ds(j * _K, _K)]], add=True)

    for b in range(_NBUF):
        gather(b, b)

    @pl.loop(0, _CH - _NBUF - _TAIL, step=_NBUF)
    def _(j):
        for b in range(_NBUF):
            wait_scatter(j + b, b)
            gather(j + b + _NBUF, b)

    for b in range(_TAIL):
        wait_scatter(_CH - _NBUF - _TAIL + b, b)
        gather(_CH - _TAIL + b, b)
    for b in range(_TAIL, _NBUF):
        wait_scatter(_CH - _NBUF - _TAIL + b, b)
    for b in range(_TAIL):
        wait_scatter(_CH - _TAIL + b, b)

    plsc.subcore_barrier()
    pltpu.sync_copy(acc.at[pl.ds(s * _RPT, _RPT)],
                    out.at[pl.ds(c * _NP + s * _RPT, _RPT)])


_sc_gather_scatter = functools.partial(
    pl.kernel,
    out_type=jax.ShapeDtypeStruct((2 * _NP, D), jnp.float32),
    mesh=_mesh,
    scratch_types=(
        [pltpu.VMEM((_PPW,), jnp.int32),
         pltpu.VMEM((_PPW,), jnp.int32)]
        + [pltpu.VMEM((_K, D), jnp.float32) for _ in range(_NBUF)]
        + [pltpu.VMEM_SHARED((_NP, D), jnp.float32)]
        + [pltpu.SemaphoreType.DMA for _ in range(_NBUF)]
    ),
)(_sc_body_gs)


_KC = 200
_CCC = _PPC // _KC
_CW = 4


def _sc_body_cnt(idxcat, ones, zeros, out, cidx, ones_v, acc, csem):
    c = lax.axis_index("c")
    s = lax.axis_index("s")
    pltpu.sync_copy(idxcat.at[pl.ds(c * NNZ + s * _PPC, _PPC)], cidx)
    pltpu.sync_copy(ones, ones_v)
    pltpu.sync_copy(zeros.at[pl.ds(s * _RPT, _RPT)], acc.at[pl.ds(s * _RPT, _RPT)])
    plsc.subcore_barrier()

    def fire(j):
        pltpu.async_copy(ones_v, acc.at[cidx.at[pl.ds(j * _KC, _KC)]],
                         csem, add=True)

    def drain_one():
        pltpu.make_async_copy(ones_v, acc.at[cidx.at[pl.ds(0, _KC)]],
                              csem).wait()

    for w in range(_CW):
        fire(w)

    @pl.loop(_CW, _CCC)
    def _(j):
        drain_one()
        fire(j)

    for _w in range(_CW):
        drain_one()

    plsc.subcore_barrier()
    pltpu.sync_copy(acc.at[pl.ds(s * _RPT, _RPT)],
                    out.at[pl.ds(c * _NP + s * _RPT, _RPT)])


_sc_counts = functools.partial(
    pl.kernel,
    out_type=jax.ShapeDtypeStruct((2 * _NP, D), jnp.float32),
    mesh=_mesh,
    scratch_types=[
        pltpu.VMEM((_PPC,), jnp.int32),
        pltpu.VMEM((_KC, D), jnp.float32),
        pltpu.VMEM_SHARED((_NP, D), jnp.float32),
        pltpu.SemaphoreType.DMA,
    ],
)(_sc_body_cnt)


_R = 1000
_RP = 1024


def _tc_mm(x, w, b2d):
    def body(x_ref, w_ref, b_ref, o_ref):
        o_ref[...] = (jnp.dot(x_ref[...], w_ref[...],
                              preferred_element_type=jnp.float32) + b_ref[...])

    return pl.pallas_call(
        body,
        grid=(N_V // _R,),
        in_specs=[
            pl.BlockSpec((_R, D), lambda i: (i, 0)),
            pl.BlockSpec((D, D), lambda i: (0, 0)),
            pl.BlockSpec((1, D), lambda i: (0, 0)),
        ],
        out_specs=pl.BlockSpec((_R, D), lambda i: (i, 0)),
        out_shape=jax.ShapeDtypeStruct((N_V, D), jnp.float32),
    )(x, w, b2d)


def _tc_combine(partials, cnts, off):
    def body(p0_ref, p1_ref, c_ref, o_ref):
        cnt = jnp.maximum(c_ref[...][:, :1], 1.0)
        o_ref[...] = (p0_ref[...] + p1_ref[...]) / cnt

    nb = _NP // _RP
    return pl.pallas_call(
        body,
        grid=(nb,),
        in_specs=[
            pl.BlockSpec((_RP, D), lambda i: (i, 0)),
            pl.BlockSpec((_RP, D), lambda i: (i + nb, 0)),
            pl.BlockSpec((_RP, D), lambda i: (i + off, 0)),
        ],
        out_specs=pl.BlockSpec((_RP, D), lambda i: (i, 0)),
        out_shape=jax.ShapeDtypeStruct((_NP, D), jnp.float32),
    )(partials, partials, cnts)


def _tc_combine_out(partials, cnts, off):
    def body(p0_ref, p1_ref, c_ref, o_ref):
        cnt = jnp.maximum(c_ref[...][:, :1], 1.0)
        o_ref[...] = (p0_ref[...] + p1_ref[...]) / cnt

    rb = 512
    nb = _NP // rb
    return pl.pallas_call(
        body,
        grid=(pl.cdiv(N_V, rb),),
        in_specs=[
            pl.BlockSpec((rb, D), lambda i: (i, 0)),
            pl.BlockSpec((rb, D), lambda i: (i + nb, 0)),
            pl.BlockSpec((rb, D), lambda i: (i + off, 0)),
        ],
        out_specs=pl.BlockSpec((rb, D), lambda i: (i, 0)),
        out_shape=jax.ShapeDtypeStruct((N_V, D), jnp.float32),
    )(partials, partials, cnts)


def _tc_combine_relu_mm(partials, cnts, off, w, b2d):
    def body(p0_ref, p1_ref, c_ref, w_ref, b_ref, o_ref):
        cnt = jnp.maximum(c_ref[...][:, :1], 1.0)
        v = (p0_ref[...] + p1_ref[...]) / cnt
        v = jnp.maximum(v, 0.0)
        o_ref[...] = (jnp.dot(v, w_ref[...],
                              preferred_element_type=jnp.float32) + b_ref[...])

    nb = _NP // _RP
    return pl.pallas_call(
        body,
        grid=(nb,),
        in_specs=[
            pl.BlockSpec((_RP, D), lambda i: (i, 0)),
            pl.BlockSpec((_RP, D), lambda i: (i + nb, 0)),
            pl.BlockSpec((_RP, D), lambda i: (i + off, 0)),
            pl.BlockSpec((D, D), lambda i: (0, 0)),
            pl.BlockSpec((1, D), lambda i: (0, 0)),
        ],
        out_specs=pl.BlockSpec((_RP, D), lambda i: (i, 0)),
        out_shape=jax.ShapeDtypeStruct((_NP, D), jnp.float32),
    )(partials, partials, cnts, w, b2d)


def kernel(X, incidence, W1, b1, W2, b2):
    v_idx = incidence[0].astype(jnp.int32)
    e_idx = incidence[1].astype(jnp.int32)
    idxcat = jnp.concatenate([e_idx, v_idx])
    zeros = jnp.zeros((_NP, D), jnp.float32)
    ones = jnp.ones((_KC, D), jnp.float32)
    b1r = b1.reshape(1, D)
    b2r = b2.reshape(1, D)

    nb = _NP // _RP
    cnt = _sc_counts(idxcat, ones, zeros)
    h1 = _tc_mm(X, W1, b1r)
    p = _sc_gather_scatter(h1, v_idx, e_idx, zeros)
    he1 = _tc_combine(p, cnt, 0)
    q = _sc_gather_scatter(he1, e_idx, v_idx, zeros)
    h2 = _tc_combine_relu_mm(q, cnt, nb, W2, b2r)
    p2 = _sc_gather_scatter(h2, v_idx, e_idx, zeros)
    he2 = _tc_combine(p2, cnt, 0)
    q2 = _sc_gather_scatter(he2, e_idx, v_idx, zeros)
    return _tc_combine_out(q2, cnt, _NP // 512)


# speedup vs baseline: 1.3119x; 1.0446x over previous
import functools

import jax
import jax.numpy as jnp
from jax import lax
from jax.experimental import pallas as pl
from jax.experimental.pallas import tpu as pltpu
from jax.experimental.pallas import tpu_sc as plsc

N_V = 10000
N_E = 10000
NNZ = 320000
D = 128

_NC = 2
_NS = 16
_NW = _NC * _NS
_K = 40
_PPW = NNZ // _NW
_PPC = NNZ // _NS
_CH = NNZ // _NW // _K
_CCH = NNZ // _NS // _K
_NP = 10240
_RPT = _NP // _NS

_mesh = plsc.VectorSubcoreMesh(core_axis_name="c", subcore_axis_name="s")


_NBUF = 5
_TAIL = _CH % _NBUF
assert _TAIL == 0


def _sc_body_gs(table, src1d, dst1d, zeros, out, sidx, didx,
                rows0, rows1, rows2, rows3, rows4,
                acc, sem0, sem1, sem2, sem3, sem4):
    rows = (rows0, rows1, rows2, rows3, rows4)
    sems = (sem0, sem1, sem2, sem3, sem4)
    c = lax.axis_index("c")
    s = lax.axis_index("s")
    wid = s * _NC + c
    pltpu.sync_copy(src1d.at[pl.ds(wid * _PPW, _PPW)], sidx)
    pltpu.sync_copy(dst1d.at[pl.ds(wid * _PPW, _PPW)], didx)
    pltpu.sync_copy(zeros.at[pl.ds(s * _RPT, _RPT)], acc.at[pl.ds(s * _RPT, _RPT)])
    plsc.subcore_barrier()

    def gather(j, b):
        pltpu.async_copy(table.at[sidx.at[pl.ds(j * _K, _K)]], rows[b], sems[b])

    def wait_scatter(j, b):
        pltpu.make_async_copy(table.at[sidx.at[pl.ds(j * _K, _K)]],
                              rows[b], sems[b]).wait()
        pltpu.sync_copy(rows[b], acc.at[didx.at[pl.ds(j * _K, _K)]], add=True)

    for b in range(_NBUF):
        gather(b, b)

    @pl.loop(0, _CH - _NBUF, step=_NBUF)
    def _(j):
        for b in range(_NBUF):
            wait_scatter(j + b, b)
            gather(j + b + _NBUF, b)

    for b in range(_NBUF):
        wait_scatter(_CH - _NBUF + b, b)

    plsc.subcore_barrier()
    pltpu.sync_copy(acc.at[pl.ds(s * _RPT, _RPT)],
                    out.at[pl.ds(c * _NP + s * _RPT, _RPT)])


_sc_gather_scatter = functools.partial(
    pl.kernel,
    out_type=jax.ShapeDtypeStruct((2 * _NP, D), jnp.float32),
    mesh=_mesh,
    scratch_types=(
        [pltpu.VMEM((_PPW,), jnp.int32),
         pltpu.VMEM((_PPW,), jnp.int32)]
        + [pltpu.VMEM((_K, D), jnp.float32) for _ in range(_NBUF)]
        + [pltpu.VMEM_SHARED((_NP, D), jnp.float32)]
        + [pltpu.SemaphoreType.DMA for _ in range(_NBUF)]
    ),
)(_sc_body_gs)


_KC = 200
_CCC = _PPC // _KC
_CW = 4


def _sc_body_cnt(idxcat, ones, zeros, out, cidx, ones_v, acc, csem):
    c = lax.axis_index("c")
    s = lax.axis_index("s")
    pltpu.sync_copy(idxcat.at[pl.ds(c * NNZ + s * _PPC, _PPC)], cidx)
    pltpu.sync_copy(ones, ones_v)
    pltpu.sync_copy(zeros.at[pl.ds(s * _RPT, _RPT)], acc.at[pl.ds(s * _RPT, _RPT)])
    plsc.subcore_barrier()

    def fire(j):
        pltpu.async_copy(ones_v, acc.at[cidx.at[pl.ds(j * _KC, _KC)]],
                         csem, add=True)

    def drain_one():
        pltpu.make_async_copy(ones_v, acc.at[cidx.at[pl.ds(0, _KC)]],
                              csem).wait()

    for w in range(_CW):
        fire(w)

    @pl.loop(_CW, _CCC)
    def _(j):
        drain_one()
        fire(j)

    for _w in range(_CW):
        drain_one()

    plsc.subcore_barrier()
    pltpu.sync_copy(acc.at[pl.ds(s * _RPT, _RPT)],
                    out.at[pl.ds(c * _NP + s * _RPT, _RPT)])


_sc_counts = functools.partial(
    pl.kernel,
    out_type=jax.ShapeDtypeStruct((2 * _NP, D), jnp.float32),
    mesh=_mesh,
    scratch_types=[
        pltpu.VMEM((_PPC,), jnp.int32),
        pltpu.VMEM((_KC, D), jnp.float32),
        pltpu.VMEM_SHARED((_NP, D), jnp.float32),
        pltpu.SemaphoreType.DMA,
    ],
)(_sc_body_cnt)


_R = 1000
_RP = 1024


def _tc_mm(x, w, b2d):
    def body(x_ref, w_ref, b_ref, o_ref):
        o_ref[...] = (jnp.dot(x_ref[...], w_ref[...],
                              preferred_element_type=jnp.float32) + b_ref[...])

    return pl.pallas_call(
        body,
        grid=(N_V // _R,),
        in_specs=[
            pl.BlockSpec((_R, D), lambda i: (i, 0)),
            pl.BlockSpec((D, D), lambda i: (0, 0)),
            pl.BlockSpec((1, D), lambda i: (0, 0)),
        ],
        out_specs=pl.BlockSpec((_R, D), lambda i: (i, 0)),
        out_shape=jax.ShapeDtypeStruct((N_V, D), jnp.float32),
    )(x, w, b2d)


def _tc_combine(partials, cnts, off):
    def body(p0_ref, p1_ref, c_ref, o_ref):
        cnt = jnp.maximum(c_ref[...][:, :1], 1.0)
        o_ref[...] = (p0_ref[...] + p1_ref[...]) / cnt

    nb = _NP // _RP
    return pl.pallas_call(
        body,
        grid=(nb,),
        in_specs=[
            pl.BlockSpec((_RP, D), lambda i: (i, 0)),
            pl.BlockSpec((_RP, D), lambda i: (i + nb, 0)),
            pl.BlockSpec((_RP, D), lambda i: (i + off, 0)),
        ],
        out_specs=pl.BlockSpec((_RP, D), lambda i: (i, 0)),
        out_shape=jax.ShapeDtypeStruct((_NP, D), jnp.float32),
    )(partials, partials, cnts)


def _tc_combine_out(partials, cnts, off):
    def body(p0_ref, p1_ref, c_ref, o_ref):
        cnt = jnp.maximum(c_ref[...][:, :1], 1.0)
        o_ref[...] = (p0_ref[...] + p1_ref[...]) / cnt

    rb = 512
    nb = _NP // rb
    return pl.pallas_call(
        body,
        grid=(pl.cdiv(N_V, rb),),
        in_specs=[
            pl.BlockSpec((rb, D), lambda i: (i, 0)),
            pl.BlockSpec((rb, D), lambda i: (i + nb, 0)),
            pl.BlockSpec((rb, D), lambda i: (i + off, 0)),
        ],
        out_specs=pl.BlockSpec((rb, D), lambda i: (i, 0)),
        out_shape=jax.ShapeDtypeStruct((N_V, D), jnp.float32),
    )(partials, partials, cnts)


def _tc_combine_relu_mm(partials, cnts, off, w, b2d):
    def body(p0_ref, p1_ref, c_ref, w_ref, b_ref, o_ref):
        cnt = jnp.maximum(c_ref[...][:, :1], 1.0)
        v = (p0_ref[...] + p1_ref[...]) / cnt
        v = jnp.maximum(v, 0.0)
        o_ref[...] = (jnp.dot(v, w_ref[...],
                              preferred_element_type=jnp.float32) + b_ref[...])

    nb = _NP // _RP
    return pl.pallas_call(
        body,
        grid=(nb,),
        in_specs=[
            pl.BlockSpec((_RP, D), lambda i: (i, 0)),
            pl.BlockSpec((_RP, D), lambda i: (i + nb, 0)),
            pl.BlockSpec((_RP, D), lambda i: (i + off, 0)),
            pl.BlockSpec((D, D), lambda i: (0, 0)),
            pl.BlockSpec((1, D), lambda i: (0, 0)),
        ],
        out_specs=pl.BlockSpec((_RP, D), lambda i: (i, 0)),
        out_shape=jax.ShapeDtypeStruct((_NP, D), jnp.float32),
    )(partials, partials, cnts, w, b2d)


def kernel(X, incidence, W1, b1, W2, b2):
    v_idx = incidence[0].astype(jnp.int32)
    e_idx = incidence[1].astype(jnp.int32)
    idxcat = jnp.concatenate([e_idx, v_idx])
    zeros = jnp.zeros((_NP, D), jnp.float32)
    ones = jnp.ones((_KC, D), jnp.float32)
    b1r = b1.reshape(1, D)
    b2r = b2.reshape(1, D)

    nb = _NP // _RP
    cnt = _sc_counts(idxcat, ones, zeros)
    h1 = _tc_mm(X, W1, b1r)
    p = _sc_gather_scatter(h1, v_idx, e_idx, zeros)
    he1 = _tc_combine(p, cnt, 0)
    q = _sc_gather_scatter(he1, e_idx, v_idx, zeros)
    h2 = _tc_combine_relu_mm(q, cnt, nb, W2, b2r)
    p2 = _sc_gather_scatter(h2, v_idx, e_idx, zeros)
    he2 = _tc_combine(p2, cnt, 0)
    q2 = _sc_gather_scatter(he2, e_idx, v_idx, zeros)
    return _tc_combine_out(q2, cnt, _NP // 512)
